# baseline (speedup 1.0000x reference)
import functools
import jax
import jax.numpy as jnp
from jax import lax
from jax.experimental import pallas as pl
from jax.experimental.pallas import tpu as pltpu, tpu_sc as plsc

EMBED_DIM = 2048
NUM_EXPERTS = 16
N_TOKENS = 16384
BLK = 2048

NC, NS, L = 2, 16, 16
NW = NC * NS
CHUNK = N_TOKENS // NW


NBUF = 4
CH = 512


def _make_logits_body(tok0, ntok):
    nst = ntok // CH

    def _logits_body(x_hbm, w_ref, b_ref, lt_ref, xbuf, sems):
        w = w_ref[...]
        b2 = b_ref[...]
        HC = CH // 2

        def start(i, slot):
            pltpu.make_async_copy(
                x_hbm.at[pl.ds(tok0 + i * CH, HC)],
                xbuf.at[slot, pl.ds(0, HC)], sems.at[slot, 0]).start()
            pltpu.make_async_copy(
                x_hbm.at[pl.ds(tok0 + i * CH + HC, HC)],
                xbuf.at[slot, pl.ds(HC, HC)], sems.at[slot, 1]).start()

        def wait(i, slot):
            pltpu.make_async_copy(
                x_hbm.at[pl.ds(tok0 + i * CH, HC)],
                xbuf.at[slot, pl.ds(0, HC)], sems.at[slot, 0]).wait()
            pltpu.make_async_copy(
                x_hbm.at[pl.ds(tok0 + i * CH + HC, HC)],
                xbuf.at[slot, pl.ds(HC, HC)], sems.at[slot, 1]).wait()

        for i in range(NBUF):
            start(i, i)

        def step(i, _):
            slot = lax.rem(i, NBUF)
            wait(i, slot)
            part = jax.lax.dot_general(
                w, xbuf[slot], (((1,), (1,)), ((), ())),
                preferred_element_type=jnp.float32)
            lt_ref[:, pl.ds(i * CH, CH)] = part + b2

            @pl.when(i + NBUF < nst)
            def _():
                start(i + NBUF, slot)

            return 0

        lax.fori_loop(0, nst, step, 0)

    return _logits_body


def _logits_t(x, W, b, tok0, ntok):
    return pl.pallas_call(
        _make_logits_body(tok0, ntok),
        in_specs=[
            pl.BlockSpec(memory_space=pl.ANY),
            pl.BlockSpec((NUM_EXPERTS, EMBED_DIM), lambda: (0, 0)),
            pl.BlockSpec((NUM_EXPERTS, 1), lambda: (0, 0)),
        ],
        out_specs=pl.BlockSpec((NUM_EXPERTS, ntok), lambda: (0, 0)),
        out_shape=jax.ShapeDtypeStruct((NUM_EXPERTS, ntok), jnp.float32),
        scratch_shapes=[
            pltpu.VMEM((NBUF, CH, EMBED_DIM), jnp.float32),
            pltpu.SemaphoreType.DMA((NBUF, 2)),
        ],
    )(x, W, b.reshape(NUM_EXPERTS, 1))


def _make_router(chunk):
    def _router(lt_hbm, gates_hbm, idx_hbm, lv, gv, iv):
        wid = lax.axis_index("s") * NC + lax.axis_index("c")
        base = wid * chunk
        pltpu.sync_copy(lt_hbm.at[:, pl.ds(base, chunk)], lv)

        def group(g, _):
            off = g * L
            m1 = lv[0, pl.ds(off, L)]
            i1 = jnp.zeros((L,), jnp.int32)
            m2 = jnp.full((L,), -jnp.inf, jnp.float32)
            i2 = jnp.zeros((L,), jnp.int32)
            for e in range(1, NUM_EXPERTS):
                v = lv[e, pl.ds(off, L)]
                ev = jnp.full((L,), e, jnp.int32)
                gt1 = v > m1
                gt2 = v > m2
                m2 = jnp.where(gt1, m1, jnp.where(gt2, v, m2))
                i2 = jnp.where(gt1, i1, jnp.where(gt2, ev, i2))
                m1 = jnp.where(gt1, v, m1)
                i1 = jnp.where(gt1, ev, i1)
            e2 = jnp.exp(m2 - m1)
            den = 1.0 + e2
            gv[0, pl.ds(off, L)] = 1.0 / den
            gv[1, pl.ds(off, L)] = e2 / den
            iv[0, pl.ds(off, L)] = i1
            iv[1, pl.ds(off, L)] = i2
            return 0

        lax.fori_loop(0, chunk // L, group, 0)
        pltpu.sync_copy(gv, gates_hbm.at[:, pl.ds(base, chunk)])
        pltpu.sync_copy(iv, idx_hbm.at[:, pl.ds(base, chunk)])

    return _router


def _route(lt):
    ntok = lt.shape[1]
    chunk = ntok // NW
    mesh = plsc.VectorSubcoreMesh(core_axis_name="c", subcore_axis_name="s")
    f = functools.partial(
        pl.kernel, mesh=mesh,
        out_type=[
            jax.ShapeDtypeStruct((2, ntok), jnp.float32),
            jax.ShapeDtypeStruct((2, ntok), jnp.int32),
        ],
        scratch_types=[
            pltpu.VMEM((NUM_EXPERTS, chunk), jnp.float32),
            pltpu.VMEM((2, chunk), jnp.float32),
            pltpu.VMEM((2, chunk), jnp.int32),
        ],
    )(_make_router(chunk))
    return f(lt)


NCHUNKS = 2
TOKC = N_TOKENS // NCHUNKS


def kernel(x, W, b):
    parts = []
    for c in range(NCHUNKS):
        lt = _logits_t(x, W, b, c * TOKC, TOKC)
        parts.append(_route(lt))
    gates_t = jnp.concatenate([p[0] for p in parts], axis=1)
    idx_t = jnp.concatenate([p[1] for p in parts], axis=1)
    return (gates_t.T, idx_t.T)


# speedup vs baseline: 1.1182x; 1.1182x over previous
import jax
import jax.numpy as jnp
from jax import lax
from jax.experimental import pallas as pl
from jax.experimental.pallas import tpu as pltpu

EMBED_DIM = 2048
NUM_EXPERTS = 16
N_TOKENS = 16384

NBUF = 4
CH = 512
NST = N_TOKENS // CH


def _gate_body(x_hbm, w_ref, b_ref, gates_ref, idx_ref, xbuf, sems):
    w = w_ref[...]
    b2 = b_ref[...]
    HC = CH // 2

    def start(i, slot):
        pltpu.make_async_copy(
            x_hbm.at[pl.ds(i * CH, HC)], xbuf.at[slot, pl.ds(0, HC)],
            sems.at[slot, 0]).start()
        pltpu.make_async_copy(
            x_hbm.at[pl.ds(i * CH + HC, HC)], xbuf.at[slot, pl.ds(HC, HC)],
            sems.at[slot, 1]).start()

    def wait(i, slot):
        pltpu.make_async_copy(
            x_hbm.at[pl.ds(i * CH, HC)], xbuf.at[slot, pl.ds(0, HC)],
            sems.at[slot, 0]).wait()
        pltpu.make_async_copy(
            x_hbm.at[pl.ds(i * CH + HC, HC)], xbuf.at[slot, pl.ds(HC, HC)],
            sems.at[slot, 1]).wait()

    for i in range(NBUF):
        start(i, i)

    def step(i, _):
        slot = lax.rem(i, NBUF)
        wait(i, slot)
        logits = jax.lax.dot_general(
            xbuf[slot], w, (((1,), (1,)), ((), ())),
            preferred_element_type=jnp.float32) + b2

        cols = jax.lax.broadcasted_iota(jnp.int32, logits.shape, 1)
        m1 = jnp.max(logits, axis=1, keepdims=True)
        i1 = jnp.min(jnp.where(logits == m1, cols, NUM_EXPERTS),
                     axis=1, keepdims=True)
        masked = jnp.where(cols == i1, -jnp.inf, logits)
        m2 = jnp.max(masked, axis=1, keepdims=True)
        i2 = jnp.min(jnp.where(masked == m2, cols, NUM_EXPERTS),
                     axis=1, keepdims=True)
        e2 = jnp.exp(m2 - m1)
        den = 1.0 + e2
        gates_ref[pl.ds(i * CH, CH), :] = jnp.concatenate(
            [1.0 / den, e2 / den], axis=1)
        idx_ref[pl.ds(i * CH, CH), :] = jnp.concatenate([i1, i2], axis=1)

        @pl.when(i + NBUF < NST)
        def _():
            start(i + NBUF, slot)

        return 0

    lax.fori_loop(0, NST, step, 0)


def kernel(x, W, b):
    return pl.pallas_call(
        _gate_body,
        in_specs=[
            pl.BlockSpec(memory_space=pl.ANY),
            pl.BlockSpec((NUM_EXPERTS, EMBED_DIM), lambda: (0, 0)),
            pl.BlockSpec((1, NUM_EXPERTS), lambda: (0, 0)),
        ],
        out_specs=[
            pl.BlockSpec((N_TOKENS, 2), lambda: (0, 0)),
            pl.BlockSpec((N_TOKENS, 2), lambda: (0, 0)),
        ],
        out_shape=[
            jax.ShapeDtypeStruct((N_TOKENS, 2), jnp.float32),
            jax.ShapeDtypeStruct((N_TOKENS, 2), jnp.int32),
        ],
        scratch_shapes=[
            pltpu.VMEM((NBUF, CH, EMBED_DIM), jnp.float32),
            pltpu.SemaphoreType.DMA((NBUF, 2)),
        ],
    )(x, W, b.reshape(1, NUM_EXPERTS))


# speedup vs baseline: 1.1393x; 1.0188x over previous
import jax
import jax.numpy as jnp
from jax import lax
from jax.experimental import pallas as pl
from jax.experimental.pallas import tpu as pltpu

EMBED_DIM = 2048
NUM_EXPERTS = 16
N_TOKENS = 16384

NBUF = 4
CH = 512
NST = N_TOKENS // CH


def _gate_body(x_hbm, w_ref, b_ref, gates_ref, idx_ref, xbuf, sems):
    w = w_ref[...]
    b2 = b_ref[...]
    HC = CH // 2

    def start(i, slot):
        pltpu.make_async_copy(
            x_hbm.at[pl.ds(i * CH, HC)], xbuf.at[slot, pl.ds(0, HC)],
            sems.at[slot, 0]).start()
        pltpu.make_async_copy(
            x_hbm.at[pl.ds(i * CH + HC, HC)], xbuf.at[slot, pl.ds(HC, HC)],
            sems.at[slot, 1]).start()

    def wait(i, slot):
        pltpu.make_async_copy(
            x_hbm.at[pl.ds(i * CH, HC)], xbuf.at[slot, pl.ds(0, HC)],
            sems.at[slot, 0]).wait()
        pltpu.make_async_copy(
            x_hbm.at[pl.ds(i * CH + HC, HC)], xbuf.at[slot, pl.ds(HC, HC)],
            sems.at[slot, 1]).wait()

    for i in range(NBUF):
        start(i, i)

    def step(i, _):
        slot = lax.rem(i, NBUF)
        wait(i, slot)
        logits = jax.lax.dot_general(
            w, xbuf[slot], (((1,), (1,)), ((), ())),
            preferred_element_type=jnp.float32) + b2

        rows = jax.lax.broadcasted_iota(jnp.int32, logits.shape, 0)
        m1 = jnp.max(logits, axis=0, keepdims=True)
        i1 = jnp.min(jnp.where(logits == m1, rows, NUM_EXPERTS),
                     axis=0, keepdims=True)
        masked = jnp.where(rows == i1, -jnp.inf, logits)
        m2 = jnp.max(masked, axis=0, keepdims=True)
        i2 = jnp.min(jnp.where(masked == m2, rows, NUM_EXPERTS),
                     axis=0, keepdims=True)
        e2 = jnp.exp(m2 - m1)
        den = 1.0 + e2
        g = jnp.concatenate([1.0 / den, e2 / den], axis=0)
        ix = jnp.concatenate([i1, i2], axis=0)
        gates_ref[pl.ds(i * CH, CH), :] = g.T
        idx_ref[pl.ds(i * CH, CH), :] = ix.T

        @pl.when(i + NBUF < NST)
        def _():
            start(i + NBUF, slot)

        return 0

    lax.fori_loop(0, NST, step, 0)


def kernel(x, W, b):
    return pl.pallas_call(
        _gate_body,
        in_specs=[
            pl.BlockSpec(memory_space=pl.ANY),
            pl.BlockSpec((NUM_EXPERTS, EMBED_DIM), lambda: (0, 0)),
            pl.BlockSpec((NUM_EXPERTS, 1), lambda: (0, 0)),
        ],
        out_specs=[
            pl.BlockSpec((N_TOKENS, 2), lambda: (0, 0)),
            pl.BlockSpec((N_TOKENS, 2), lambda: (0, 0)),
        ],
        out_shape=[
            jax.ShapeDtypeStruct((N_TOKENS, 2), jnp.float32),
            jax.ShapeDtypeStruct((N_TOKENS, 2), jnp.int32),
        ],
        scratch_shapes=[
            pltpu.VMEM((NBUF, CH, EMBED_DIM), jnp.float32),
            pltpu.SemaphoreType.DMA((NBUF, 2)),
        ],
    )(x, W, b.reshape(NUM_EXPERTS, 1))


# speedup vs baseline: 1.5694x; 1.3776x over previous
import jax
import jax.numpy as jnp
from jax import lax
from jax.experimental import pallas as pl
from jax.experimental.pallas import tpu as pltpu

EMBED_DIM = 2048
NUM_EXPERTS = 16
N_TOKENS = 16384

NBUF = 4
CH = 512
NST = N_TOKENS // CH


def _gate_body(x_hbm, w_ref, b_ref, gates_ref, idx_ref, xbuf, sems):
    w = w_ref[...]
    b2 = b_ref[...]
    HC = CH // 2

    def start(i, slot):
        pltpu.make_async_copy(
            x_hbm.at[pl.ds(i * CH, HC)], xbuf.at[slot, pl.ds(0, HC)],
            sems.at[slot, 0]).start()
        pltpu.make_async_copy(
            x_hbm.at[pl.ds(i * CH + HC, HC)], xbuf.at[slot, pl.ds(HC, HC)],
            sems.at[slot, 1]).start()

    def wait(i, slot):
        pltpu.make_async_copy(
            x_hbm.at[pl.ds(i * CH, HC)], xbuf.at[slot, pl.ds(0, HC)],
            sems.at[slot, 0]).wait()
        pltpu.make_async_copy(
            x_hbm.at[pl.ds(i * CH + HC, HC)], xbuf.at[slot, pl.ds(HC, HC)],
            sems.at[slot, 1]).wait()

    for i in range(NBUF):
        start(i, i)

    def step(i, _):
        slot = lax.rem(i, NBUF)
        wait(i, slot)
        logits = jax.lax.dot_general(
            w, xbuf[slot], (((1,), (1,)), ((), ())),
            preferred_element_type=jnp.float32) + b2

        rows = jax.lax.broadcasted_iota(jnp.int32, logits.shape, 0)
        m1 = jnp.max(logits, axis=0, keepdims=True)
        i1 = jnp.min(jnp.where(logits == m1, rows, NUM_EXPERTS),
                     axis=0, keepdims=True)
        masked = jnp.where(rows == i1, -jnp.inf, logits)
        m2 = jnp.max(masked, axis=0, keepdims=True)
        i2 = jnp.min(jnp.where(masked == m2, rows, NUM_EXPERTS),
                     axis=0, keepdims=True)
        e2 = jnp.exp(m2 - m1)
        den = 1.0 + e2
        g = jnp.concatenate([1.0 / den, e2 / den], axis=0)
        ix = jnp.concatenate([i1, i2], axis=0)
        gates_ref[:, pl.ds(i * CH, CH)] = g
        idx_ref[:, pl.ds(i * CH, CH)] = ix

        @pl.when(i + NBUF < NST)
        def _():
            start(i + NBUF, slot)

        return 0

    lax.fori_loop(0, NST, step, 0)


def kernel(x, W, b):
    gates_t, idx_t = pl.pallas_call(
        _gate_body,
        in_specs=[
            pl.BlockSpec(memory_space=pl.ANY),
            pl.BlockSpec((NUM_EXPERTS, EMBED_DIM), lambda: (0, 0)),
            pl.BlockSpec((NUM_EXPERTS, 1), lambda: (0, 0)),
        ],
        out_specs=[
            pl.BlockSpec((2, N_TOKENS), lambda: (0, 0)),
            pl.BlockSpec((2, N_TOKENS), lambda: (0, 0)),
        ],
        out_shape=[
            jax.ShapeDtypeStruct((2, N_TOKENS), jnp.float32),
            jax.ShapeDtypeStruct((2, N_TOKENS), jnp.int32),
        ],
        scratch_shapes=[
            pltpu.VMEM((NBUF, CH, EMBED_DIM), jnp.float32),
            pltpu.SemaphoreType.DMA((NBUF, 2)),
        ],
    )(x, W, b.reshape(NUM_EXPERTS, 1))
    return (gates_t.T, idx_t.T)
